# baseline (speedup 1.0000x reference)
import functools

import jax
import jax.numpy as jnp
from jax.experimental import pallas as pl
from jax.experimental.pallas import tpu as pltpu


def _gcn_step(a_ref, x_ref, m_ref, w1_ref, b1_ref, w2_ref, b2_ref, ph_ref):
    af = a_ref[0].astype(jnp.bfloat16)
    x = x_ref[0]
    m_row = m_ref[0]
    m_col = jnp.transpose(m_row)

    dn = (((0,), (0,)), ((), ()))
    t1 = jax.lax.dot_general(af, m_col.astype(jnp.bfloat16), dn,
                             preferred_element_type=jnp.float32)
    deg = m_col * (t1 + 1.0)
    dinv = jax.lax.rsqrt(jnp.maximum(deg, 1e-12))
    md = m_col * dinv
    mdd = md * dinv

    hp1 = jnp.dot(x, w1_ref[...], preferred_element_type=jnp.float32)
    s1 = jax.lax.dot_general(af, (md * hp1).astype(jnp.bfloat16), dn,
                             preferred_element_type=jnp.float32)
    h1 = jax.nn.relu(md * s1 + mdd * hp1 + b1_ref[...])

    hp2 = jnp.dot(h1, w2_ref[...], preferred_element_type=jnp.float32)
    s2 = jax.lax.dot_general(af, (md * hp2).astype(jnp.bfloat16), dn,
                             preferred_element_type=jnp.float32)
    ph_ref[0] = m_col * (md * s2 + mdd * hp2 + b2_ref[...])


def _lstm_attn_step(ph_ref, wih_ref, whh_ref, b_ref, aw_ref, out_ref,
                    *, n_nodes, hid, nchain):
    wih = wih_ref[...].astype(jnp.bfloat16)
    whh = whh_ref[...].astype(jnp.bfloat16)
    b = b_ref[...]
    aw = aw_ref[...]
    SUB = 8
    tc = ph_ref.shape[2]

    def body(i, carry):
        hs, cs = map(list, carry)
        xq = ph_ref[:, pl.ds(i * SUB, SUB)]
        rows = []
        for k in range(SUB):
            ss = []
            for q in range(nchain):
                g = (jnp.dot(xq[q, k].astype(jnp.bfloat16), wih,
                             preferred_element_type=jnp.float32)
                     + jnp.dot(hs[q].astype(jnp.bfloat16), whh,
                               preferred_element_type=jnp.float32) + b)
                ig = jax.nn.sigmoid(g[:, 0:hid])
                fg = jax.nn.sigmoid(g[:, hid:2 * hid])
                gg = jnp.tanh(g[:, 2 * hid:3 * hid])
                og = jax.nn.sigmoid(g[:, 3 * hid:4 * hid])
                cs[q] = fg * cs[q] + ig * gg
                hs[q] = og * jnp.tanh(cs[q])
                ss.append(jnp.sum(hs[q] * aw, axis=1, keepdims=True))
            mx = ss[0]
            for q in range(1, nchain):
                mx = jnp.maximum(mx, ss[q])
            mx = jnp.max(mx, axis=0, keepdims=True)
            num = None
            den = None
            for q in range(nchain):
                e = jnp.exp(ss[q] - mx)
                nq = jnp.sum(hs[q] * e, axis=0, keepdims=True)
                dq = jnp.sum(e, axis=0, keepdims=True)
                num = nq if num is None else num + nq
                den = dq if den is None else den + dq
            rows.append(num / den)
        out_ref[pl.ds(i * SUB, SUB), :] = jnp.concatenate(rows, axis=0)
        return tuple(hs), tuple(cs)

    z = jnp.zeros((tc, hid), jnp.float32)
    init = (tuple(z for _ in range(nchain)), tuple(z for _ in range(nchain)))
    jax.lax.fori_loop(0, n_nodes // SUB, body, init)


def kernel(ego_mask_batch, big_batch_positions, big_batched_adjacency_pruned,
           gcn1_W, gcn1_b, gcn2_W, gcn2_b,
           lstm_W_ih, lstm_W_hh, lstm_b_ih, lstm_b_hh, attn_W, attn_b):
    T, N, F = big_batch_positions.shape
    bsz, _, max_nodes = ego_mask_batch.shape
    hid = gcn1_W.shape[1]
    G = lstm_W_ih.shape[0]

    mask = (jnp.transpose(ego_mask_batch, (1, 0, 2))
            .reshape(T, 1, N).astype(jnp.float32))

    ph = pl.pallas_call(
        _gcn_step,
        grid=(T,),
        in_specs=[
            pl.BlockSpec((1, N, N), lambda t: (t, 0, 0)),
            pl.BlockSpec((1, N, F), lambda t: (t, 0, 0)),
            pl.BlockSpec((1, 1, N), lambda t: (t, 0, 0)),
            pl.BlockSpec((F, hid), lambda t: (0, 0)),
            pl.BlockSpec((1, hid), lambda t: (0, 0)),
            pl.BlockSpec((hid, hid), lambda t: (0, 0)),
            pl.BlockSpec((1, hid), lambda t: (0, 0)),
        ],
        out_specs=pl.BlockSpec((1, N, hid), lambda t: (t, 0, 0)),
        out_shape=jax.ShapeDtypeStruct((T, N, hid), jnp.float32),
        compiler_params=pltpu.CompilerParams(
            dimension_semantics=("arbitrary",)),
    )(big_batched_adjacency_pruned, big_batch_positions, mask,
      gcn1_W, gcn1_b.reshape(1, hid), gcn2_W, gcn2_b.reshape(1, hid))

    nchain = 4
    tc = T // nchain
    ph4 = jnp.transpose(ph.reshape(nchain, tc, N, hid), (0, 2, 1, 3))

    bias = (lstm_b_ih + lstm_b_hh).reshape(1, G)
    xout = pl.pallas_call(
        functools.partial(_lstm_attn_step, n_nodes=N, hid=hid, nchain=nchain),
        grid=(1,),
        in_specs=[
            pl.BlockSpec((nchain, N, tc, hid), lambda i: (0, 0, 0, 0)),
            pl.BlockSpec((hid, G), lambda i: (0, 0)),
            pl.BlockSpec((hid, G), lambda i: (0, 0)),
            pl.BlockSpec((1, G), lambda i: (0, 0)),
            pl.BlockSpec((1, hid), lambda i: (0, 0)),
        ],
        out_specs=pl.BlockSpec((N, hid), lambda i: (0, 0)),
        out_shape=jax.ShapeDtypeStruct((N, hid), jnp.float32),
        compiler_params=pltpu.CompilerParams(
            dimension_semantics=("arbitrary",)),
    )(ph4, jnp.transpose(lstm_W_ih), jnp.transpose(lstm_W_hh), bias, attn_W)

    return xout.reshape(bsz, max_nodes, hid)
